# baseline (speedup 1.0000x reference)
import functools
import jax
import jax.numpy as jnp
from jax import lax
from jax.experimental import pallas as pl
from jax.experimental.pallas import tpu as pltpu
from jax.experimental.pallas import tpu_sc as plsc


def _argmin_parts(g, iota_f):
    m = jnp.min(g, axis=1, keepdims=True)
    cand = jnp.where(g == m, iota_f, jnp.float32(g.shape[1]))
    idx = jnp.min(cand, axis=1, keepdims=True)
    return m, idx


def _body(f_ref, w_ref, b_ref, ebd_ref, embt_ref,
          tout_ref, idxp_ref, loss_ref, *, loss_scale, d, k):
    i = pl.program_id(0)
    x = f_ref[...]
    TP = jnp.dot(x, w_ref[...], preferred_element_type=jnp.float32) + b_ref[...]

    ebd = ebd_ref[...]
    embT = embt_ref[...]
    es = jnp.sum(ebd * ebd, axis=0, keepdims=True)
    cross = jnp.dot(TP, ebd, preferred_element_type=jnp.float32)

    T = TP[:, :d]
    Pv = TP[:, d:]
    xsT = jnp.sum(T * T, axis=1, keepdims=True)
    distT = (xsT - 2.0 * cross[:, :k]) + es[:, :k]
    xsP = jnp.sum(Pv * Pv, axis=1, keepdims=True)
    distP = (xsP - 2.0 * cross[:, k:]) + es[:, k:]

    iota_f = jax.lax.broadcasted_iota(jnp.int32, distT.shape, 1).astype(jnp.float32)
    mT, idxT = _argmin_parts(distT, iota_f)
    mP, idxP = _argmin_parts(distP, iota_f)
    encT = (iota_f == idxT).astype(jnp.float32)
    idxp_ref[...] = idxP.astype(jnp.int32)

    e = jnp.exp(mT - distT)
    lse = jnp.log(jnp.sum(e, axis=1, keepdims=True))
    w = (mT - distT) - lse

    tout_ref[...] = jnp.dot(0.5 * (w + encT), embT,
                            preferred_element_type=jnp.float32)

    partial = ((jnp.sum(mT) + jnp.sum(mP)) * loss_scale).reshape(1, 1)

    @pl.when(i == 0)
    def _():
        loss_ref[...] = partial

    @pl.when(i != 0)
    def _():
        loss_ref[...] = loss_ref[...] + partial


def _sc_gather(table, idx, N, D):
    info = plsc.get_sparse_core_info()
    NW = info.num_cores * info.num_subcores
    b_per_w = N // NW
    CH = 128
    n_ch = b_per_w // CH
    mesh = plsc.VectorSubcoreMesh(core_axis_name="c", subcore_axis_name="s")

    @functools.partial(
        pl.kernel, mesh=mesh,
        out_type=jax.ShapeDtypeStruct((N, D), jnp.float32),
        scratch_types=[
            pltpu.VMEM((n_ch, CH), jnp.int32),
            pltpu.VMEM((CH, D), jnp.float32),
            pltpu.SemaphoreType.DMA,
        ],
    )
    def sc_k(table_hbm, idx_hbm, out_hbm, idx_v, rows, sem):
        wid = lax.axis_index("s") * info.num_cores + lax.axis_index("c")
        base = wid * b_per_w
        pltpu.sync_copy(idx_hbm.at[wid], idx_v)
        for c in range(n_ch):
            pltpu.async_copy(table_hbm.at[idx_v.at[c]], rows, sem).wait()
            pltpu.sync_copy(rows, out_hbm.at[pl.ds(base + c * CH, CH)])

    return sc_k(table, idx.reshape(NW, n_ch, CH))


def kernel(f, W_T, b_T, W_P, b_P, emb_T, emb_P):
    B, L, E = f.shape
    N = B * L
    D = W_T.shape[1]
    K = emb_T.shape[0]
    BT = 4096
    ff = f.reshape(N, E)
    W = jnp.concatenate([W_T, W_P], axis=1)
    b = jnp.concatenate([b_T, b_P]).reshape(1, 2 * D)
    z = jnp.zeros((D, K), jnp.float32)
    ebd = jnp.concatenate(
        [jnp.concatenate([emb_T.T, z], axis=1),
         jnp.concatenate([z, emb_P.T], axis=1)], axis=0)
    loss_scale = 1.25 / (N * D)

    grid = (N // BT,)
    const_spec = lambda shape: pl.BlockSpec(shape, lambda i: (0, 0))
    T_out, idx_P, loss = pl.pallas_call(
        functools.partial(_body, loss_scale=loss_scale, d=D, k=K),
        grid=grid,
        in_specs=[
            pl.BlockSpec((BT, E), lambda i: (i, 0)),
            const_spec((E, 2 * D)),
            const_spec((1, 2 * D)),
            const_spec((2 * D, 2 * K)),
            const_spec(emb_T.shape),
        ],
        out_specs=[
            pl.BlockSpec((BT, D), lambda i: (i, 0)),
            pl.BlockSpec((BT, 1), lambda i: (i, 0)),
            pl.BlockSpec((1, 1), lambda i: (0, 0)),
        ],
        out_shape=[
            jax.ShapeDtypeStruct((N, D), jnp.float32),
            jax.ShapeDtypeStruct((N, 1), jnp.int32),
            jax.ShapeDtypeStruct((1, 1), jnp.float32),
        ],
    )(ff, W, b, ebd, emb_T)

    P_out = _sc_gather(emb_P, idx_P.reshape(N), N, D)

    return T_out.reshape(B, L, D), P_out.reshape(B, L, D), loss[0, 0]


# speedup vs baseline: 2.3435x; 2.3435x over previous
import functools
import jax
import jax.numpy as jnp
from jax.experimental import pallas as pl


def _argmin_parts(g, iota_f):
    m = jnp.min(g, axis=1, keepdims=True)
    cand = jnp.where(g == m, iota_f, jnp.float32(g.shape[1]))
    idx = jnp.min(cand, axis=1, keepdims=True)
    enc = (iota_f == idx).astype(jnp.float32)
    return m, enc


def _body(f_ref, w_ref, b_ref, ebd_ref, embt_ref, embp_ref,
          tout_ref, pout_ref, loss_ref, *, loss_scale, d, k):
    i = pl.program_id(0)
    x = f_ref[...]
    TP = jnp.dot(x, w_ref[...], preferred_element_type=jnp.float32) + b_ref[...]

    ebd = ebd_ref[...]
    embT = embt_ref[...]
    embP = embp_ref[...]
    es = jnp.sum(ebd * ebd, axis=0, keepdims=True)
    cross = jnp.dot(TP, ebd, preferred_element_type=jnp.float32)

    T = TP[:, :d]
    Pv = TP[:, d:]
    xsT = jnp.sum(T * T, axis=1, keepdims=True)
    distT = (xsT - 2.0 * cross[:, :k]) + es[:, :k]
    xsP = jnp.sum(Pv * Pv, axis=1, keepdims=True)
    distP = (xsP - 2.0 * cross[:, k:]) + es[:, k:]

    iota_f = jax.lax.broadcasted_iota(jnp.int32, distT.shape, 1).astype(jnp.float32)
    mT, encT = _argmin_parts(distT, iota_f)
    mP, encP = _argmin_parts(distP, iota_f)

    e = jnp.exp(mT - distT)
    lse = jnp.log(jnp.sum(e, axis=1, keepdims=True))
    w = (mT - distT) - lse

    tout_ref[...] = jnp.dot(0.5 * (w + encT), embT,
                            preferred_element_type=jnp.float32)
    pout_ref[...] = jnp.dot(encP, embP, preferred_element_type=jnp.float32)

    partial = ((jnp.sum(mT) + jnp.sum(mP)) * loss_scale).reshape(1, 1)

    @pl.when(i == 0)
    def _():
        loss_ref[...] = partial

    @pl.when(i != 0)
    def _():
        loss_ref[...] = loss_ref[...] + partial


def kernel(f, W_T, b_T, W_P, b_P, emb_T, emb_P):
    B, L, E = f.shape
    N = B * L
    D = W_T.shape[1]
    K = emb_T.shape[0]
    BT = 4096
    ff = f.reshape(N, E)
    W = jnp.concatenate([W_T, W_P], axis=1)
    b = jnp.concatenate([b_T, b_P]).reshape(1, 2 * D)
    z = jnp.zeros((D, K), jnp.float32)
    ebd = jnp.concatenate(
        [jnp.concatenate([emb_T.T, z], axis=1),
         jnp.concatenate([z, emb_P.T], axis=1)], axis=0)
    loss_scale = 1.25 / (N * D)

    grid = (N // BT,)
    const_spec = lambda shape: pl.BlockSpec(shape, lambda i: (0, 0))
    T_out, P_out, loss = pl.pallas_call(
        functools.partial(_body, loss_scale=loss_scale, d=D, k=K),
        grid=grid,
        in_specs=[
            pl.BlockSpec((BT, E), lambda i: (i, 0)),
            const_spec((E, 2 * D)),
            const_spec((1, 2 * D)),
            const_spec((2 * D, 2 * K)),
            const_spec(emb_T.shape),
            const_spec(emb_P.shape),
        ],
        out_specs=[
            pl.BlockSpec((BT, D), lambda i: (i, 0)),
            pl.BlockSpec((BT, D), lambda i: (i, 0)),
            pl.BlockSpec((1, 1), lambda i: (0, 0)),
        ],
        out_shape=[
            jax.ShapeDtypeStruct((N, D), jnp.float32),
            jax.ShapeDtypeStruct((N, D), jnp.float32),
            jax.ShapeDtypeStruct((1, 1), jnp.float32),
        ],
    )(ff, W, b, ebd, emb_T, emb_P)

    return T_out.reshape(B, L, D), P_out.reshape(B, L, D), loss[0, 0]
